# Initial kernel scaffold; baseline (speedup 1.0000x reference)
import jax
import jax.numpy as jnp
from jax.experimental import pallas as pl


def kernel(hidden_states, router_w, router_b, wq, wk, wv, wo, g1, g2, wg, wu, wd):
    raise NotImplementedError("write your pallas kernel here")


# speedup vs baseline: 1.6171x; 1.6171x over previous
import functools

import jax
import jax.numpy as jnp
from jax.experimental import pallas as pl
from jax.experimental.pallas import tpu as pltpu

B, S, D = 4, 4096, 768
DFF = 3072
EPS = 1e-05
K = max(1, int(0.5 * S))

ROUTER_BLK = 1024
TOK_BLK = 256


def _router_body(h_ref, rw_ref, rb_ref, w_ref):
    logits = jnp.dot(h_ref[...], rw_ref[...], preferred_element_type=jnp.float32)
    w_ref[...] = jax.nn.sigmoid(logits + rb_ref[0])


def _rank_body(wrow_ref, wcol_ref, mask_ref):
    w_row = wrow_ref[0]
    w_col = wcol_ref[0]
    cnt = jnp.zeros((1, S), jnp.float32)
    CH = 512
    for c in range(S // CH):
        wc = w_col[c * CH:(c + 1) * CH, :]
        gt = (wc > w_row).astype(jnp.float32)
        cnt = cnt + jnp.sum(gt, axis=0, keepdims=True)
    mask_ref[0] = (cnt < float(K)).astype(jnp.float32)


def _block_body(x_ref, m_ref, wv_ref, wo_ref, g1_ref, g2_ref,
                wg_ref, wu_ref, wd_ref, o_ref, w2_ref):
    @pl.when(pl.program_id(0) == 0)
    def _():
        w2 = jnp.dot(wv_ref[...], wo_ref[...],
                     preferred_element_type=jnp.float32)
        w2_ref[...] = w2.astype(jnp.bfloat16)

    x = x_ref[...]
    n1 = x * jax.lax.rsqrt(jnp.mean(x * x, axis=-1, keepdims=True) + EPS)
    n1 = (n1 * g1_ref[...]).astype(jnp.bfloat16)
    attn = jnp.dot(n1, w2_ref[...], preferred_element_type=jnp.float32)
    r = x + attn
    n2 = r * jax.lax.rsqrt(jnp.mean(r * r, axis=-1, keepdims=True) + EPS)
    n2 = (n2 * g2_ref[...]).astype(jnp.bfloat16)
    gg = jnp.dot(n2, wg_ref[...], preferred_element_type=jnp.float32)
    uu = jnp.dot(n2, wu_ref[...], preferred_element_type=jnp.float32)
    h = ((gg * jax.nn.sigmoid(gg)) * uu).astype(jnp.bfloat16)
    f = jnp.dot(h, wd_ref[...], preferred_element_type=jnp.float32)
    y = r + f
    m = m_ref[...]
    o_ref[...] = x + m * (y - x)


def kernel(hidden_states, router_w, router_b, wq, wk, wv, wo, g1, g2, wg, wu, wd):
    del wq, wk
    hid = hidden_states.reshape(B * S, D)

    weights = pl.pallas_call(
        _router_body,
        out_shape=jax.ShapeDtypeStruct((B * S, 1), jnp.float32),
        grid=(B * S // ROUTER_BLK,),
        in_specs=[
            pl.BlockSpec((ROUTER_BLK, D), lambda i: (i, 0)),
            pl.BlockSpec((D, 1), lambda i: (0, 0)),
            pl.BlockSpec(memory_space=pltpu.SMEM),
        ],
        out_specs=pl.BlockSpec((ROUTER_BLK, 1), lambda i: (i, 0)),
    )(hid, router_w, router_b)

    w_row3 = weights.reshape(B, 1, S)
    w_col3 = weights.reshape(B, S, 1)

    mask3 = pl.pallas_call(
        _rank_body,
        out_shape=jax.ShapeDtypeStruct((B, 1, S), jnp.float32),
        grid=(B,),
        in_specs=[
            pl.BlockSpec((1, 1, S), lambda b: (b, 0, 0)),
            pl.BlockSpec((1, S, 1), lambda b: (b, 0, 0)),
        ],
        out_specs=pl.BlockSpec((1, 1, S), lambda b: (b, 0, 0)),
    )(w_row3, w_col3)
    mask = mask3.reshape(B * S, 1)

    wv_b = wv.astype(jnp.bfloat16)
    wo_b = wo.astype(jnp.bfloat16)
    wg_b = wg.astype(jnp.bfloat16)
    wu_b = wu.astype(jnp.bfloat16)
    wd_b = wd.astype(jnp.bfloat16)
    g1r = g1.reshape(1, D)
    g2r = g2.reshape(1, D)

    out = pl.pallas_call(
        _block_body,
        out_shape=jax.ShapeDtypeStruct((B * S, D), jnp.float32),
        grid=(B * S // TOK_BLK,),
        in_specs=[
            pl.BlockSpec((TOK_BLK, D), lambda i: (i, 0)),
            pl.BlockSpec((TOK_BLK, 1), lambda i: (i, 0)),
            pl.BlockSpec((D, D), lambda i: (0, 0)),
            pl.BlockSpec((D, D), lambda i: (0, 0)),
            pl.BlockSpec((1, D), lambda i: (0, 0)),
            pl.BlockSpec((1, D), lambda i: (0, 0)),
            pl.BlockSpec((D, DFF), lambda i: (0, 0)),
            pl.BlockSpec((D, DFF), lambda i: (0, 0)),
            pl.BlockSpec((DFF, D), lambda i: (0, 0)),
        ],
        out_specs=pl.BlockSpec((TOK_BLK, D), lambda i: (i, 0)),
        scratch_shapes=[pltpu.VMEM((D, D), jnp.bfloat16)],
        compiler_params=pltpu.CompilerParams(
            dimension_semantics=("arbitrary",)),
    )(hid, mask, wv_b, wo_b, g1r, g2r, wg_b, wu_b, wd_b)

    return out.reshape(B, S, D)
